# Initial kernel scaffold; baseline (speedup 1.0000x reference)
import jax
import jax.numpy as jnp
from jax.experimental import pallas as pl


def kernel(stm_indices, nstm_indices, table, input_bias, W, b):
    raise NotImplementedError("write your pallas kernel here")


# speedup vs baseline: 1.3796x; 1.3796x over previous
import functools

import jax
import jax.numpy as jnp
from jax import lax
from jax.experimental import pallas as pl
from jax.experimental.pallas import tpu as pltpu
from jax.experimental.pallas import tpu_sc as plsc

BATCH = 16384
NIDX = 32
FEATS = 768
HID = 1024
NBKT = 8
PAD = FEATS

NCORES = 2
NSUB = 16
LANES = 16
DSLICE = HID // NSUB
NCH = DSLICE // LANES
SAMP_PER_CORE = BATCH // NCORES
CHUNK = 512
NCHUNK = SAMP_PER_CORE // CHUNK
STRIP = SAMP_PER_CORE // NSUB

QSCALE = float(2 ** 19)
INV_QSCALE = 1.0 / QSCALE


def _nnue_body(stm_hbm, nstm_hbm, tbl_hbm, bias_hbm, wst_hbm, wns_hbm,
               b_hbm, out_hbm,
               tbl_v, stm_v, nstm_v, wst_v, wns_v, bias_v, b_v,
               part_v, red_v, fin_v, shared):
    c = lax.axis_index("c")
    s = lax.axis_index("s")
    base = c * SAMP_PER_CORE

    pltpu.sync_copy(tbl_hbm.at[s], tbl_v)
    pltpu.sync_copy(wst_hbm.at[s], wst_v)
    pltpu.sync_copy(wns_hbm.at[s], wns_v)
    pltpu.sync_copy(bias_hbm.at[s], bias_v)
    pltpu.sync_copy(b_hbm, b_v)

    zero32 = jnp.zeros((2 * LANES,), jnp.int16)
    for cc in range(NCH // 2):
        tbl_v[PAD, pl.ds(2 * LANES * cc, 2 * LANES)] = zero32

    bscale = jnp.where(s == 0, jnp.float32(1.0), jnp.float32(0.0))
    lane_iota = lax.iota(jnp.int32, LANES)
    ones16 = jnp.ones((LANES,), jnp.int32)
    zeros16i = jnp.zeros((LANES,), jnp.int32)
    zeros16f = jnp.zeros((LANES,), jnp.float32)
    bvec = b_v[pl.ds(0, LANES)]
    bias_ch = [bias_v[pl.ds(LANES * cc, LANES)] for cc in range(NCH)]

    def chunk_body(k, carry):
        cb = base + k * CHUNK
        pltpu.sync_copy(stm_hbm.at[pl.ds(cb, CHUNK)], stm_v)
        pltpu.sync_copy(nstm_hbm.at[pl.ds(cb, CHUNK)], nstm_v)

        def group_body(g, carry2):
            def samp_body(u, reg):
                i = g * LANES + u
                ivs0 = stm_v[i, pl.ds(0, LANES)]
                ivs1 = stm_v[i, pl.ds(LANES, LANES)]
                ivn0 = nstm_v[i, pl.ds(0, LANES)]
                ivn1 = nstm_v[i, pl.ds(LANES, LANES)]
                cntv = (plsc.all_reduce_population_count(ivs0 != PAD)
                        + plsc.all_reduce_population_count(ivs1 != PAD))
                bktv = jnp.maximum((cntv - 2) >> 2, zeros16i)
                bkt = bktv[0]

                accs = [zeros16i for _ in range(2 * NCH)]
                for r in range(LANES):
                    for h, iv in ((0, ivs0), (0, ivs1),
                                  (NCH, ivn0), (NCH, ivn1)):
                        ridx = iv[r]
                        for w in range(NCH // 2):
                            sl = pl.ds(2 * LANES * w, 2 * LANES)
                            pa, pb = plsc.unpack(
                                tbl_v[ridx, sl],
                                format=plsc.PackFormat.INTERLEAVED,
                                preferred_element_type=jnp.int32)
                            accs[h + 2 * w] = accs[h + 2 * w] + pa
                            accs[h + 2 * w + 1] = accs[h + 2 * w + 1] + pb
                dot = zeros16f
                for cc in range(NCH):
                    sl = pl.ds(LANES * cc, LANES)
                    bl = bias_ch[cc]
                    a_s = accs[cc].astype(jnp.float32) * INV_QSCALE
                    a_n = accs[NCH + cc].astype(jnp.float32) * INV_QSCALE
                    e_s = jnp.clip(a_s + bl, 0.0, 1.0)
                    e_n = jnp.clip(a_n + bl, 0.0, 1.0)
                    dot = dot + (e_s * e_s) * wst_v[bkt, sl]
                    dot = dot + (e_n * e_n) * wns_v[bkt, sl]
                bval = jnp.sum(jnp.where(lane_iota == bktv, bvec, zeros16f))
                res = jnp.sum(dot) + bscale * bval
                return jnp.where(lane_iota == u, jnp.full((LANES,), res), reg)

            reg = lax.fori_loop(0, LANES, samp_body, zeros16f)
            part_v[pl.ds(k * CHUNK + g * LANES, LANES)] = reg
            return carry2

        lax.fori_loop(0, CHUNK // LANES, group_body, 0)
        return carry

    lax.fori_loop(0, NCHUNK, chunk_body, 0)

    pltpu.sync_copy(part_v, shared.at[s])
    plsc.subcore_barrier()
    pltpu.sync_copy(shared.at[:, pl.ds(s * STRIP, STRIP)], red_v)
    for j in range(STRIP // LANES):
        sl = pl.ds(LANES * j, LANES)
        acc = red_v[0, sl]
        for t in range(1, NSUB):
            acc = acc + red_v[t, sl]
        fin_v[sl] = acc
    pltpu.sync_copy(fin_v, out_hbm.at[pl.ds(base + s * STRIP, STRIP)])


_nnue_call = functools.partial(
    pl.kernel,
    out_type=jax.ShapeDtypeStruct((BATCH,), jnp.float32),
    mesh=plsc.VectorSubcoreMesh(core_axis_name="c", subcore_axis_name="s"),
    compiler_params=pltpu.CompilerParams(needs_layout_passes=False,
                                         use_tc_tiling_on_sc=False),
    scratch_types=[
        pltpu.VMEM((FEATS + 1, DSLICE), jnp.int16),
        pltpu.VMEM((CHUNK, NIDX), jnp.int32),
        pltpu.VMEM((CHUNK, NIDX), jnp.int32),
        pltpu.VMEM((NBKT, DSLICE), jnp.float32),
        pltpu.VMEM((NBKT, DSLICE), jnp.float32),
        pltpu.VMEM((DSLICE,), jnp.float32),
        pltpu.VMEM((2 * NBKT,), jnp.float32),
        pltpu.VMEM((SAMP_PER_CORE,), jnp.float32),
        pltpu.VMEM((NSUB, STRIP), jnp.float32),
        pltpu.VMEM((STRIP,), jnp.float32),
        pltpu.VMEM_SHARED((NSUB, SAMP_PER_CORE), jnp.float32),
    ],
)(_nnue_body)


_PERM = tuple(list(range(0, 32, 2)) + list(range(1, 32, 2))
              + list(range(32, 64, 2)) + list(range(33, 64, 2)))


def kernel(stm_indices, nstm_indices, table, input_bias, W, b):
    stm = stm_indices.astype(jnp.int32)
    nstm = nstm_indices.astype(jnp.int32)
    tbl_q = jnp.round(table.astype(jnp.float32) * QSCALE).astype(jnp.int16)
    tbl_r = tbl_q.reshape(FEATS + 1, NSUB, DSLICE)
    tbl_r = tbl_r.transpose(1, 0, 2)
    w32 = W.astype(jnp.float32)
    perm = jnp.array(_PERM, dtype=jnp.int32)
    wst_r = (w32[:, :HID].reshape(NBKT, NSUB, DSLICE)[:, :, perm]
             .transpose(1, 0, 2))
    wns_r = (w32[:, HID:].reshape(NBKT, NSUB, DSLICE)[:, :, perm]
             .transpose(1, 0, 2))
    bias_r = input_bias.astype(jnp.float32).reshape(NSUB, DSLICE)[:, perm]
    b_pad = jnp.pad(b.astype(jnp.float32), (0, NBKT))
    out = _nnue_call(stm, nstm, tbl_r, bias_r, wst_r, wns_r, b_pad)
    return out.reshape(BATCH, 1)


# speedup vs baseline: 16.7350x; 12.1304x over previous
import functools

import jax
import jax.numpy as jnp
from jax import lax
from jax.experimental import pallas as pl
from jax.experimental.pallas import tpu as pltpu
from jax.experimental.pallas import tpu_sc as plsc

BATCH = 16384
NIDX = 32
FEATS = 768
HID = 1024
NBKT = 8
PAD = FEATS

NCORES = 2
NSUB = 16
LANES = 16
DSLICE = HID // NSUB
NCH = DSLICE // LANES
CHUNK = 512

QSCALE = float(2 ** 19)
INV_QSCALE = 1.0 / QSCALE

NSC_SPLIT = 0


def _make_sc_call(nsc):
    samp_per_core = nsc // NCORES
    nchunk = samp_per_core // CHUNK
    strip = samp_per_core // NSUB

    def _nnue_body(stm_hbm, nstm_hbm, tbl_hbm, bias_hbm, wst_hbm, wns_hbm,
                   b_hbm, out_hbm,
                   tbl_v, stm_v, nstm_v, wst_v, wns_v, bias_v, b_v,
                   part_v, red_v, fin_v, shared):
        c = lax.axis_index("c")
        s = lax.axis_index("s")
        base = c * samp_per_core

        pltpu.sync_copy(tbl_hbm.at[s], tbl_v)
        pltpu.sync_copy(wst_hbm.at[s], wst_v)
        pltpu.sync_copy(wns_hbm.at[s], wns_v)
        pltpu.sync_copy(bias_hbm.at[s], bias_v)
        pltpu.sync_copy(b_hbm, b_v)

        zero16 = jnp.zeros((LANES,), jnp.float32)
        for cc in range(NCH):
            tbl_v[PAD, pl.ds(LANES * cc, LANES)] = zero16

        bscale = jnp.where(s == 0, jnp.float32(1.0), jnp.float32(0.0))
        lane_iota = lax.iota(jnp.int32, LANES)
        zeros16i = jnp.zeros((LANES,), jnp.int32)
        zeros16f = jnp.zeros((LANES,), jnp.float32)
        bvec = b_v[pl.ds(0, LANES)]
        bias_ch = [bias_v[pl.ds(LANES * cc, LANES)] for cc in range(NCH)]

        def chunk_body(k, carry):
            cb = base + k * CHUNK
            pltpu.sync_copy(stm_hbm.at[pl.ds(cb, CHUNK)], stm_v)
            pltpu.sync_copy(nstm_hbm.at[pl.ds(cb, CHUNK)], nstm_v)

            def group_body(g, carry2):
                def samp_body(u, reg):
                    i = g * LANES + u
                    ivs0 = stm_v[i, pl.ds(0, LANES)]
                    ivs1 = stm_v[i, pl.ds(LANES, LANES)]
                    ivn0 = nstm_v[i, pl.ds(0, LANES)]
                    ivn1 = nstm_v[i, pl.ds(LANES, LANES)]
                    cntv = (plsc.all_reduce_population_count(ivs0 != PAD)
                            + plsc.all_reduce_population_count(ivs1 != PAD))
                    bktv = jnp.maximum((cntv - 2) >> 2, zeros16i)
                    bkt = bktv[0]

                    accs = [zeros16f for _ in range(2 * NCH)]
                    for r in range(LANES):
                        s0 = ivs0[r]
                        s1 = ivs1[r]
                        n0 = ivn0[r]
                        n1 = ivn1[r]
                        for cc in range(NCH):
                            sl = pl.ds(LANES * cc, LANES)
                            accs[cc] = (accs[cc]
                                        + tbl_v[s0, sl] + tbl_v[s1, sl])
                            accs[NCH + cc] = (accs[NCH + cc]
                                              + tbl_v[n0, sl] + tbl_v[n1, sl])
                    dot = zeros16f
                    for cc in range(NCH):
                        sl = pl.ds(LANES * cc, LANES)
                        bl = bias_ch[cc]
                        e_s = jnp.clip(accs[cc] + bl, 0.0, 1.0)
                        e_n = jnp.clip(accs[NCH + cc] + bl, 0.0, 1.0)
                        dot = dot + (e_s * e_s) * wst_v[bkt, sl]
                        dot = dot + (e_n * e_n) * wns_v[bkt, sl]
                    bval = jnp.sum(jnp.where(lane_iota == bktv, bvec,
                                             zeros16f))
                    res = jnp.sum(dot) + bscale * bval
                    return jnp.where(lane_iota == u, jnp.full((LANES,), res),
                                     reg)

                reg = lax.fori_loop(0, LANES, samp_body, zeros16f)
                part_v[pl.ds(k * CHUNK + g * LANES, LANES)] = reg
                return carry2

            lax.fori_loop(0, CHUNK // LANES, group_body, 0)
            return carry

        lax.fori_loop(0, nchunk, chunk_body, 0)

        pltpu.sync_copy(part_v, shared.at[s])
        plsc.subcore_barrier()
        pltpu.sync_copy(shared.at[:, pl.ds(s * strip, strip)], red_v)
        for j in range(strip // LANES):
            sl = pl.ds(LANES * j, LANES)
            acc = red_v[0, sl]
            for t in range(1, NSUB):
                acc = acc + red_v[t, sl]
            fin_v[sl] = acc
        pltpu.sync_copy(fin_v, out_hbm.at[pl.ds(base + s * strip, strip)])

    return functools.partial(
        pl.kernel,
        out_type=jax.ShapeDtypeStruct((nsc,), jnp.float32),
        mesh=plsc.VectorSubcoreMesh(core_axis_name="c", subcore_axis_name="s"),
        compiler_params=pltpu.CompilerParams(needs_layout_passes=False,
                                             use_tc_tiling_on_sc=False),
        scratch_types=[
            pltpu.VMEM((FEATS + 1, DSLICE), jnp.float32),
            pltpu.VMEM((CHUNK, NIDX), jnp.int32),
            pltpu.VMEM((CHUNK, NIDX), jnp.int32),
            pltpu.VMEM((NBKT, DSLICE), jnp.float32),
            pltpu.VMEM((NBKT, DSLICE), jnp.float32),
            pltpu.VMEM((DSLICE,), jnp.float32),
            pltpu.VMEM((2 * NBKT,), jnp.float32),
            pltpu.VMEM((samp_per_core,), jnp.float32),
            pltpu.VMEM((NSUB, strip), jnp.float32),
            pltpu.VMEM((strip,), jnp.float32),
            pltpu.VMEM_SHARED((NSUB, samp_per_core), jnp.float32),
        ],
    )(_nnue_body)


_sc_call = _make_sc_call(NSC_SPLIT) if NSC_SPLIT else None


TCB = 512


def _tc_body(stmt_ref, nstmt_ref, thi_ref, tlo_ref, bias_ref,
             wstt_ref, wnst_ref, b_ref, out_ref):
    fi = lax.broadcasted_iota(jnp.int32, (TCB, FEATS + 1), 1).astype(jnp.int16)
    one = jnp.ones((TCB, FEATS + 1), jnp.bfloat16)
    zero = jnp.zeros((TCB, FEATS + 1), jnp.bfloat16)

    def one_hot(idx_ref):
        oh = zero
        pc = jnp.zeros((TCB,), jnp.int32)
        for a in range(NIDX):
            col = idx_ref[a, :][:, None]
            oh = oh + jnp.where(col == fi, one, zero)
            pc = pc + (col[:, 0] == PAD).astype(jnp.int32)
        return oh, pc

    oh_s, pc_s = one_hot(stmt_ref)
    oh_n, _ = one_hot(nstmt_ref)

    def bag(oh):
        hi = jnp.dot(oh, thi_ref[...], preferred_element_type=jnp.float32)
        lo = jnp.dot(oh, tlo_ref[...], preferred_element_type=jnp.float32)
        acc = hi * 256.0 + lo
        e = jnp.clip(acc * INV_QSCALE + bias_ref[...], 0.0, 1.0)
        return e * e

    e_s = bag(oh_s)
    e_n = bag(oh_n)
    out_all = (jnp.dot(e_s, wstt_ref[...], preferred_element_type=jnp.float32)
               + jnp.dot(e_n, wnst_ref[...],
                         preferred_element_type=jnp.float32))
    cnt = NIDX - pc_s
    bkt = jnp.maximum((cnt - 2) >> 2, 0)
    res = jnp.zeros((TCB,), jnp.float32)
    for j in range(NBKT):
        sel = (bkt == j)
        res = res + jnp.where(sel, out_all[:, j] + b_ref[j], 0.0)
    out_ref[...] = res


def _tc_call(stmt, nstmt, thi, tlo, bias2d, wstt, wnst, b, n):
    ntc = n // TCB
    return pl.pallas_call(
        _tc_body,
        grid=(ntc,),
        in_specs=[
            pl.BlockSpec((NIDX, TCB), lambda i: (0, i)),
            pl.BlockSpec((NIDX, TCB), lambda i: (0, i)),
            pl.BlockSpec((FEATS + 1, HID), lambda i: (0, 0)),
            pl.BlockSpec((FEATS + 1, HID), lambda i: (0, 0)),
            pl.BlockSpec((1, HID), lambda i: (0, 0)),
            pl.BlockSpec((HID, NBKT), lambda i: (0, 0)),
            pl.BlockSpec((HID, NBKT), lambda i: (0, 0)),
            pl.BlockSpec(memory_space=pltpu.SMEM),
        ],
        out_specs=pl.BlockSpec((TCB,), lambda i: (i,)),
        out_shape=jax.ShapeDtypeStruct((n,), jnp.float32),
    )(stmt, nstmt, thi, tlo, bias2d, wstt, wnst, b)


def kernel(stm_indices, nstm_indices, table, input_bias, W, b):
    stm = stm_indices.astype(jnp.int32)
    nstm = nstm_indices.astype(jnp.int32)
    t32 = table.astype(jnp.float32)
    w32 = W.astype(jnp.float32)
    bias32 = input_bias.astype(jnp.float32)
    b32 = b.astype(jnp.float32)

    outs = []
    if NSC_SPLIT:
        tbl_r = t32.reshape(FEATS + 1, NSUB, DSLICE).transpose(1, 0, 2)
        wst_r = w32[:, :HID].reshape(NBKT, NSUB, DSLICE).transpose(1, 0, 2)
        wns_r = w32[:, HID:].reshape(NBKT, NSUB, DSLICE).transpose(1, 0, 2)
        bias_r = bias32.reshape(NSUB, DSLICE)
        b_pad = jnp.pad(b32, (0, NBKT))
        outs.append(_sc_call(stm[:NSC_SPLIT], nstm[:NSC_SPLIT], tbl_r,
                             bias_r, wst_r, wns_r, b_pad))

    if NSC_SPLIT < BATCH:
        qf = jnp.round(t32 * QSCALE).at[PAD].set(0.0)
        hi = jnp.round(qf * (1.0 / 256.0))
        lo = qf - 256.0 * hi
        thi = hi.astype(jnp.bfloat16)
        tlo = lo.astype(jnp.bfloat16)
        stmt = stm[NSC_SPLIT:].T.astype(jnp.int16)
        nstmt = nstm[NSC_SPLIT:].T.astype(jnp.int16)
        bias2d = bias32.reshape(1, HID)
        wstt = w32[:, :HID].T
        wnst = w32[:, HID:].T
        outs.append(_tc_call(stmt, nstmt, thi, tlo, bias2d, wstt, wnst, b32,
                             BATCH - NSC_SPLIT))

    out = outs[0] if len(outs) == 1 else jnp.concatenate(outs)
    return out.reshape(BATCH, 1)


# speedup vs baseline: 17.0294x; 1.0176x over previous
import functools

import jax
import jax.numpy as jnp
from jax import lax
from jax.experimental import pallas as pl
from jax.experimental.pallas import tpu as pltpu
from jax.experimental.pallas import tpu_sc as plsc

BATCH = 16384
NIDX = 32
FEATS = 768
HID = 1024
NBKT = 8
PAD = FEATS

NCORES = 2
NSUB = 16
LANES = 16
DSLICE = HID // NSUB
NCH = DSLICE // LANES
CHUNK = 512

QSCALE = float(2 ** 19)
INV_QSCALE = 1.0 / QSCALE

NSC_SPLIT = 2048


def _make_sc_call(nsc):
    samp_per_core = nsc // NCORES
    nchunk = samp_per_core // CHUNK
    strip = samp_per_core // NSUB

    def _nnue_body(stm_hbm, nstm_hbm, tbl_hbm, bias_hbm, wst_hbm, wns_hbm,
                   b_hbm, out_hbm,
                   tbl_v, stm_v, nstm_v, wst_v, wns_v, bias_v, b_v,
                   part_v, red_v, fin_v, shared):
        c = lax.axis_index("c")
        s = lax.axis_index("s")
        base = c * samp_per_core

        pltpu.sync_copy(tbl_hbm.at[s], tbl_v)
        pltpu.sync_copy(wst_hbm.at[s], wst_v)
        pltpu.sync_copy(wns_hbm.at[s], wns_v)
        pltpu.sync_copy(bias_hbm.at[s], bias_v)
        pltpu.sync_copy(b_hbm, b_v)

        zero16 = jnp.zeros((LANES,), jnp.float32)
        for cc in range(NCH):
            tbl_v[PAD, pl.ds(LANES * cc, LANES)] = zero16

        bscale = jnp.where(s == 0, jnp.float32(1.0), jnp.float32(0.0))
        lane_iota = lax.iota(jnp.int32, LANES)
        zeros16i = jnp.zeros((LANES,), jnp.int32)
        zeros16f = jnp.zeros((LANES,), jnp.float32)
        bvec = b_v[pl.ds(0, LANES)]
        bias_ch = [bias_v[pl.ds(LANES * cc, LANES)] for cc in range(NCH)]

        def chunk_body(k, carry):
            cb = base + k * CHUNK
            pltpu.sync_copy(stm_hbm.at[pl.ds(cb, CHUNK)], stm_v)
            pltpu.sync_copy(nstm_hbm.at[pl.ds(cb, CHUNK)], nstm_v)

            def group_body(g, carry2):
                def samp_body(u, reg):
                    i = g * LANES + u
                    ivs0 = stm_v[i, pl.ds(0, LANES)]
                    ivs1 = stm_v[i, pl.ds(LANES, LANES)]
                    ivn0 = nstm_v[i, pl.ds(0, LANES)]
                    ivn1 = nstm_v[i, pl.ds(LANES, LANES)]
                    cntv = (plsc.all_reduce_population_count(ivs0 != PAD)
                            + plsc.all_reduce_population_count(ivs1 != PAD))
                    bktv = jnp.maximum((cntv - 2) >> 2, zeros16i)
                    bkt = bktv[0]

                    accs = [zeros16f for _ in range(2 * NCH)]
                    for r in range(LANES):
                        s0 = ivs0[r]
                        s1 = ivs1[r]
                        n0 = ivn0[r]
                        n1 = ivn1[r]
                        for cc in range(NCH):
                            sl = pl.ds(LANES * cc, LANES)
                            accs[cc] = (accs[cc]
                                        + tbl_v[s0, sl] + tbl_v[s1, sl])
                            accs[NCH + cc] = (accs[NCH + cc]
                                              + tbl_v[n0, sl] + tbl_v[n1, sl])
                    dot = zeros16f
                    for cc in range(NCH):
                        sl = pl.ds(LANES * cc, LANES)
                        bl = bias_ch[cc]
                        e_s = jnp.clip(accs[cc] + bl, 0.0, 1.0)
                        e_n = jnp.clip(accs[NCH + cc] + bl, 0.0, 1.0)
                        dot = dot + (e_s * e_s) * wst_v[bkt, sl]
                        dot = dot + (e_n * e_n) * wns_v[bkt, sl]
                    bval = jnp.sum(jnp.where(lane_iota == bktv, bvec,
                                             zeros16f))
                    res = jnp.sum(dot) + bscale * bval
                    return jnp.where(lane_iota == u, jnp.full((LANES,), res),
                                     reg)

                reg = lax.fori_loop(0, LANES, samp_body, zeros16f)
                part_v[pl.ds(k * CHUNK + g * LANES, LANES)] = reg
                return carry2

            lax.fori_loop(0, CHUNK // LANES, group_body, 0)
            return carry

        lax.fori_loop(0, nchunk, chunk_body, 0)

        pltpu.sync_copy(part_v, shared.at[s])
        plsc.subcore_barrier()
        pltpu.sync_copy(shared.at[:, pl.ds(s * strip, strip)], red_v)
        for j in range(strip // LANES):
            sl = pl.ds(LANES * j, LANES)
            acc = red_v[0, sl]
            for t in range(1, NSUB):
                acc = acc + red_v[t, sl]
            fin_v[sl] = acc
        pltpu.sync_copy(fin_v, out_hbm.at[pl.ds(base + s * strip, strip)])

    return functools.partial(
        pl.kernel,
        out_type=jax.ShapeDtypeStruct((nsc,), jnp.float32),
        mesh=plsc.VectorSubcoreMesh(core_axis_name="c", subcore_axis_name="s"),
        compiler_params=pltpu.CompilerParams(needs_layout_passes=False,
                                             use_tc_tiling_on_sc=False),
        scratch_types=[
            pltpu.VMEM((FEATS + 1, DSLICE), jnp.float32),
            pltpu.VMEM((CHUNK, NIDX), jnp.int32),
            pltpu.VMEM((CHUNK, NIDX), jnp.int32),
            pltpu.VMEM((NBKT, DSLICE), jnp.float32),
            pltpu.VMEM((NBKT, DSLICE), jnp.float32),
            pltpu.VMEM((DSLICE,), jnp.float32),
            pltpu.VMEM((2 * NBKT,), jnp.float32),
            pltpu.VMEM((samp_per_core,), jnp.float32),
            pltpu.VMEM((NSUB, strip), jnp.float32),
            pltpu.VMEM((strip,), jnp.float32),
            pltpu.VMEM_SHARED((NSUB, samp_per_core), jnp.float32),
        ],
    )(_nnue_body)


_sc_call = _make_sc_call(NSC_SPLIT) if NSC_SPLIT else None


TCB = 512


def _tc_body(stmt_ref, nstmt_ref, thi_ref, tlo_ref, bias_ref,
             wstt_ref, wnst_ref, b_ref, out_ref):
    fi = lax.broadcasted_iota(jnp.int32, (TCB, FEATS + 1), 1).astype(jnp.int16)
    one = jnp.ones((TCB, FEATS + 1), jnp.bfloat16)
    zero = jnp.zeros((TCB, FEATS + 1), jnp.bfloat16)

    def one_hot(idx_ref):
        oh = zero
        pc = jnp.zeros((TCB,), jnp.int32)
        for a in range(NIDX):
            col = idx_ref[a, :][:, None]
            oh = oh + jnp.where(col == fi, one, zero)
            pc = pc + (col[:, 0] == PAD).astype(jnp.int32)
        return oh, pc

    oh_s, pc_s = one_hot(stmt_ref)
    oh_n, _ = one_hot(nstmt_ref)

    def bag(oh):
        hi = jnp.dot(oh, thi_ref[...], preferred_element_type=jnp.float32)
        lo = jnp.dot(oh, tlo_ref[...], preferred_element_type=jnp.float32)
        acc = hi * 256.0 + lo
        e = jnp.clip(acc * INV_QSCALE + bias_ref[...], 0.0, 1.0)
        return e * e

    e_s = bag(oh_s)
    e_n = bag(oh_n)
    out_all = (jnp.dot(e_s, wstt_ref[...], preferred_element_type=jnp.float32)
               + jnp.dot(e_n, wnst_ref[...],
                         preferred_element_type=jnp.float32))
    cnt = NIDX - pc_s
    bkt = jnp.maximum((cnt - 2) >> 2, 0)
    res = jnp.zeros((TCB,), jnp.float32)
    for j in range(NBKT):
        sel = (bkt == j)
        res = res + jnp.where(sel, out_all[:, j] + b_ref[j], 0.0)
    out_ref[...] = res


def _tc_call(stmt, nstmt, thi, tlo, bias2d, wstt, wnst, b, n):
    ntc = n // TCB
    return pl.pallas_call(
        _tc_body,
        grid=(ntc,),
        in_specs=[
            pl.BlockSpec((NIDX, TCB), lambda i: (0, i)),
            pl.BlockSpec((NIDX, TCB), lambda i: (0, i)),
            pl.BlockSpec((FEATS + 1, HID), lambda i: (0, 0)),
            pl.BlockSpec((FEATS + 1, HID), lambda i: (0, 0)),
            pl.BlockSpec((1, HID), lambda i: (0, 0)),
            pl.BlockSpec((HID, NBKT), lambda i: (0, 0)),
            pl.BlockSpec((HID, NBKT), lambda i: (0, 0)),
            pl.BlockSpec(memory_space=pltpu.SMEM),
        ],
        out_specs=pl.BlockSpec((TCB,), lambda i: (i,)),
        out_shape=jax.ShapeDtypeStruct((n,), jnp.float32),
    )(stmt, nstmt, thi, tlo, bias2d, wstt, wnst, b)


def kernel(stm_indices, nstm_indices, table, input_bias, W, b):
    stm = stm_indices.astype(jnp.int32)
    nstm = nstm_indices.astype(jnp.int32)
    t32 = table.astype(jnp.float32)
    w32 = W.astype(jnp.float32)
    bias32 = input_bias.astype(jnp.float32)
    b32 = b.astype(jnp.float32)

    outs = []
    if NSC_SPLIT:
        tbl_r = t32.reshape(FEATS + 1, NSUB, DSLICE).transpose(1, 0, 2)
        wst_r = w32[:, :HID].reshape(NBKT, NSUB, DSLICE).transpose(1, 0, 2)
        wns_r = w32[:, HID:].reshape(NBKT, NSUB, DSLICE).transpose(1, 0, 2)
        bias_r = bias32.reshape(NSUB, DSLICE)
        b_pad = jnp.pad(b32, (0, NBKT))
        outs.append(_sc_call(stm[:NSC_SPLIT], nstm[:NSC_SPLIT], tbl_r,
                             bias_r, wst_r, wns_r, b_pad))

    if NSC_SPLIT < BATCH:
        qf = jnp.round(t32 * QSCALE).at[PAD].set(0.0)
        hi = jnp.round(qf * (1.0 / 256.0))
        lo = qf - 256.0 * hi
        thi = hi.astype(jnp.bfloat16)
        tlo = lo.astype(jnp.bfloat16)
        stmt = stm[NSC_SPLIT:].T.astype(jnp.int16)
        nstmt = nstm[NSC_SPLIT:].T.astype(jnp.int16)
        bias2d = bias32.reshape(1, HID)
        wstt = w32[:, :HID].T
        wnst = w32[:, HID:].T
        outs.append(_tc_call(stmt, nstmt, thi, tlo, bias2d, wstt, wnst, b32,
                             BATCH - NSC_SPLIT))

    out = outs[0] if len(outs) == 1 else jnp.concatenate(outs)
    return out.reshape(BATCH, 1)
